# Initial kernel scaffold; baseline (speedup 1.0000x reference)
import jax
import jax.numpy as jnp
from jax import lax
from jax.experimental import pallas as pl
from jax.experimental.pallas import tpu as pltpu
from jax.experimental.pallas import tpu_sc as plsc

_N = 10000
_E = 320000
_IN = 128
_H = 16
_C = 2

_CHUNK = 128
_NSUB = 16
_NCORE = 2
_CPT = 160
_EP = _CHUNK * _CPT * _NSUB
_PADE = _EP - _E
_ROWS_PT = 632
_NP = _ROWS_PT * _NSUB
_TRASH = _N


def _make_agg(with_deg: bool):
    mesh = plsc.VectorSubcoreMesh(core_axis_name="c", subcore_axis_name="s")
    n_out = 2 if with_deg else 1
    out_type = [jax.ShapeDtypeStruct((_NCORE, _NP, _H), jnp.float32)] * n_out
    scratch = [
        pltpu.VMEM((_CPT, _CHUNK), jnp.int32),
        pltpu.VMEM((_CPT, _CHUNK), jnp.int32),
        pltpu.VMEM((_CHUNK, _H), jnp.float32),
        pltpu.VMEM((_CHUNK, _H), jnp.float32),
        pltpu.VMEM((_CHUNK, _H), jnp.float32),
        pltpu.VMEM_SHARED((_NP, _H), jnp.float32),
    ]
    if with_deg:
        scratch.append(pltpu.VMEM_SHARED((_NP, _H), jnp.float32))
    scratch += [pltpu.SemaphoreType.DMA, pltpu.SemaphoreType.DMA]

    def body(table, src, dst, zrows, ones, *refs):
        if with_deg:
            (msg_out, deg_out, src_v, dst_v, rows0, rows1, ones_v,
             acc, accd, sem0, sem1) = refs
        else:
            (msg_out, src_v, dst_v, rows0, rows1, ones_v,
             acc, sem0, sem1) = refs
            deg_out = accd = None
        cid = lax.axis_index("c")
        sid = lax.axis_index("s")
        wid = cid * _NSUB + sid
        c0 = wid * _CPT
        pltpu.sync_copy(src.at[pl.ds(c0, _CPT)], src_v)
        pltpu.sync_copy(dst.at[pl.ds(c0, _CPT)], dst_v)
        if with_deg:
            pltpu.sync_copy(ones, ones_v)
        r0 = sid * _ROWS_PT
        pltpu.sync_copy(zrows, acc.at[pl.ds(r0, _ROWS_PT)])
        if with_deg:
            pltpu.sync_copy(zrows, accd.at[pl.ds(r0, _ROWS_PT)])
        plsc.subcore_barrier()

        def fire(j, buf, sem):
            pltpu.async_copy(table.at[src_v.at[j]], buf, sem)

        def wait(j, buf, sem):
            pltpu.make_async_copy(table.at[src_v.at[j]], buf, sem).wait()

        def scat(j, buf):
            pltpu.sync_copy(buf, acc.at[dst_v.at[j]], add=True)
            if with_deg:
                pltpu.sync_copy(ones_v, accd.at[dst_v.at[j]], add=True)

        fire(0, rows0, sem0)

        def step(i, carry):
            g = 2 * i
            fire(g + 1, rows1, sem1)
            wait(g, rows0, sem0)
            scat(g, rows0)

            @pl.when(g + 2 < _CPT)
            def _():
                fire(g + 2, rows0, sem0)

            wait(g + 1, rows1, sem1)
            scat(g + 1, rows1)
            return carry

        lax.fori_loop(0, _CPT // 2, step, 0)

        plsc.subcore_barrier()
        pltpu.sync_copy(acc.at[pl.ds(r0, _ROWS_PT)],
                        msg_out.at[cid, pl.ds(r0, _ROWS_PT)])
        if with_deg:
            pltpu.sync_copy(accd.at[pl.ds(r0, _ROWS_PT)],
                            deg_out.at[cid, pl.ds(r0, _ROWS_PT)])

    return pl.kernel(body, mesh=mesh, out_type=out_type,
                     scratch_types=scratch)


_AGG_L1 = _make_agg(with_deg=True)
_AGG_L2 = _make_agg(with_deg=False)


def _linear1(x, W1f, b1f, W1i, b1i):
    blk = 1000

    def body(x_ref, wf_ref, bf_ref, wi_ref, bi_ref, out_ref):
        xb = x_ref[...]
        out_ref[0] = jnp.dot(xb, wf_ref[...],
                             preferred_element_type=jnp.float32) + bf_ref[...]
        out_ref[1] = jnp.dot(xb, wi_ref[...],
                             preferred_element_type=jnp.float32) + bi_ref[...]

    return pl.pallas_call(
        body,
        grid=(_N // blk,),
        in_specs=[
            pl.BlockSpec((blk, _IN), lambda i: (i, 0)),
            pl.BlockSpec((_IN, _H), lambda i: (0, 0)),
            pl.BlockSpec((1, _H), lambda i: (0, 0)),
            pl.BlockSpec((_IN, _H), lambda i: (0, 0)),
            pl.BlockSpec((1, _H), lambda i: (0, 0)),
        ],
        out_specs=pl.BlockSpec((2, blk, _H), lambda i: (0, i, 0)),
        out_shape=jax.ShapeDtypeStruct((2, _N, _H), jnp.float32),
    )(x, W1f, b1f.reshape(1, _H), W1i, b1i.reshape(1, _H))


def _h1_combine(msg, deg):
    def body(m_ref, d_ref, out_ref):
        h = (m_ref[0] / jnp.maximum(d_ref[0], 1.0)
             + m_ref[1] / jnp.maximum(d_ref[1], 1.0))
        out_ref[...] = jnp.where(h >= 0, h, 0.01 * h)

    return pl.pallas_call(
        body,
        out_shape=jax.ShapeDtypeStruct((_NP, _H), jnp.float32),
    )(msg, deg)


def _h2_combine(msg2, deg, W2f, b2f, W2i, b2i):
    def body(m_ref, d_ref, wf_ref, bf_ref, wi_ref, bi_ref, out_ref):
        d0 = d_ref[0]
        d1 = d_ref[1]
        m0 = m_ref[0] / jnp.maximum(d0, 1.0)
        m1 = m_ref[1] / jnp.maximum(d1, 1.0)
        out = jnp.dot(m0, wf_ref[...], preferred_element_type=jnp.float32)
        out = out + jnp.dot(m1, wi_ref[...],
                            preferred_element_type=jnp.float32)
        out = out + jnp.minimum(d0[:, 0:1], 1.0) * bf_ref[...]
        out = out + jnp.minimum(d1[:, 0:1], 1.0) * bi_ref[...]
        out_ref[...] = out

    return pl.pallas_call(
        body,
        out_shape=jax.ShapeDtypeStruct((_NP, _C), jnp.float32),
    )(msg2, deg, W2f, b2f.reshape(1, _C), W2i, b2i.reshape(1, _C))


def kernel(x, edge_index_follows, edge_index_interacts,
           W1_f, b1_f, W1_i, b1_i, W2_f, b2_f, W2_i, b2_i):
    src_f, dst_f = edge_index_follows[0], edge_index_follows[1]
    src_i, dst_i = edge_index_interacts[0], edge_index_interacts[1]

    padz = jnp.zeros((_PADE,), jnp.int32)
    padt = jnp.full((_PADE,), _TRASH, jnp.int32)
    src1 = jnp.concatenate([src_f, padz, src_i + _N, padz]).reshape(-1, _CHUNK)
    src2 = jnp.concatenate([src_f, padz, src_i, padz]).reshape(-1, _CHUNK)
    dsts = jnp.concatenate([dst_f, padt, dst_i, padt]).reshape(-1, _CHUNK)
    zrows = jnp.zeros((_ROWS_PT, _H), jnp.float32)
    ones = jnp.ones((_CHUNK, _H), jnp.float32)

    wh = _linear1(x, W1_f, b1_f, W1_i, b1_i)
    table1 = wh.reshape(2 * _N, _H)
    msg1, deg = _AGG_L1(table1, src1, dsts, zrows, ones)
    h1p = _h1_combine(msg1, deg)
    out2 = _AGG_L2(h1p, src2, dsts, zrows, ones)
    msg2 = out2[0] if isinstance(out2, (tuple, list)) else out2
    h2p = _h2_combine(msg2, deg, W2_f, b2_f, W2_i, b2_i)
    return (h2p[:_N], h1p[:_N])


# speedup vs baseline: 16.1986x; 16.1986x over previous
import jax
import jax.numpy as jnp
from jax import lax
from jax.experimental import pallas as pl
from jax.experimental.pallas import tpu as pltpu
from jax.experimental.pallas import tpu_sc as plsc

_N = 10000
_E = 320000
_IN = 128
_H = 16
_C = 2

_CHUNK = 128
_NSUB = 16
_NCORE = 2
_CPT = 160
_EP = _CHUNK * _CPT * _NSUB
_PADE = _EP - _E
_ROWS_PT = 632
_NP = _ROWS_PT * _NSUB
_TRASH = _N


def _make_agg(with_deg: bool):
    mesh = plsc.VectorSubcoreMesh(core_axis_name="c", subcore_axis_name="s")
    n_out = 2 if with_deg else 1
    out_type = [jax.ShapeDtypeStruct((_NCORE, _NP, _H), jnp.float32)] * n_out
    scratch = [
        pltpu.VMEM((_CPT, _CHUNK), jnp.int32),
        pltpu.VMEM((_CPT, _CHUNK), jnp.int32),
        pltpu.VMEM((_CHUNK, _H), jnp.float32),
        pltpu.VMEM((_CHUNK, _H), jnp.float32),
        pltpu.VMEM((_CHUNK, _H), jnp.float32),
        pltpu.VMEM_SHARED((_NP, _H), jnp.float32),
    ]
    if with_deg:
        scratch.append(pltpu.VMEM_SHARED((_NP, _H), jnp.float32))
    scratch += [pltpu.SemaphoreType.DMA, pltpu.SemaphoreType.DMA]

    def body(table, src, dst, zrows, ones, *refs):
        if with_deg:
            (msg_out, deg_out, src_v, dst_v, rows0, rows1, ones_v,
             acc, accd, sem0, sem1) = refs
        else:
            (msg_out, src_v, dst_v, rows0, rows1, ones_v,
             acc, sem0, sem1) = refs
            deg_out = accd = None
        cid = lax.axis_index("c")
        sid = lax.axis_index("s")
        wid = cid * _NSUB + sid
        c0 = wid * _CPT
        pltpu.sync_copy(src.at[pl.ds(c0, _CPT)], src_v)
        pltpu.sync_copy(dst.at[pl.ds(c0, _CPT)], dst_v)
        if with_deg:
            pltpu.sync_copy(ones, ones_v)
        r0 = sid * _ROWS_PT
        pltpu.sync_copy(zrows, acc.at[pl.ds(r0, _ROWS_PT)])
        if with_deg:
            pltpu.sync_copy(zrows, accd.at[pl.ds(r0, _ROWS_PT)])
        plsc.subcore_barrier()

        def fire(j, buf, sem):
            pltpu.async_copy(table.at[src_v.at[j]], buf, sem)

        def wait(j, buf, sem):
            pltpu.make_async_copy(table.at[src_v.at[j]], buf, sem).wait()

        def scat(j, buf):
            pltpu.sync_copy(buf, acc.at[dst_v.at[j]], add=True)
            if with_deg:
                pltpu.sync_copy(ones_v, accd.at[dst_v.at[j]], add=True)

        fire(0, rows0, sem0)

        def step(i, carry):
            g = 2 * i
            fire(g + 1, rows1, sem1)
            wait(g, rows0, sem0)
            scat(g, rows0)

            @pl.when(g + 2 < _CPT)
            def _():
                fire(g + 2, rows0, sem0)

            wait(g + 1, rows1, sem1)
            scat(g + 1, rows1)
            return carry

        lax.fori_loop(0, _CPT // 2, step, 0)

        plsc.subcore_barrier()
        pltpu.sync_copy(acc.at[pl.ds(r0, _ROWS_PT)],
                        msg_out.at[cid, pl.ds(r0, _ROWS_PT)])
        if with_deg:
            pltpu.sync_copy(accd.at[pl.ds(r0, _ROWS_PT)],
                            deg_out.at[cid, pl.ds(r0, _ROWS_PT)])

    return pl.kernel(body, mesh=mesh, out_type=out_type,
                     scratch_types=scratch,
                     compiler_params=pltpu.CompilerParams(
                         use_tc_tiling_on_sc=False))


_AGG_L1 = _make_agg(with_deg=True)
_AGG_L2 = _make_agg(with_deg=False)


def _linear1(x, W1f, b1f, W1i, b1i):
    blk = 1000

    def body(x_ref, wf_ref, bf_ref, wi_ref, bi_ref, out_ref):
        xb = x_ref[...]
        out_ref[0] = jnp.dot(xb, wf_ref[...],
                             preferred_element_type=jnp.float32) + bf_ref[...]
        out_ref[1] = jnp.dot(xb, wi_ref[...],
                             preferred_element_type=jnp.float32) + bi_ref[...]

    return pl.pallas_call(
        body,
        grid=(_N // blk,),
        in_specs=[
            pl.BlockSpec((blk, _IN), lambda i: (i, 0)),
            pl.BlockSpec((_IN, _H), lambda i: (0, 0)),
            pl.BlockSpec((1, _H), lambda i: (0, 0)),
            pl.BlockSpec((_IN, _H), lambda i: (0, 0)),
            pl.BlockSpec((1, _H), lambda i: (0, 0)),
        ],
        out_specs=pl.BlockSpec((2, blk, _H), lambda i: (0, i, 0)),
        out_shape=jax.ShapeDtypeStruct((2, _N, _H), jnp.float32),
    )(x, W1f, b1f.reshape(1, _H), W1i, b1i.reshape(1, _H))


def _h1_combine(msg, deg):
    def body(m_ref, d_ref, out_ref):
        h = (m_ref[0] / jnp.maximum(d_ref[0], 1.0)
             + m_ref[1] / jnp.maximum(d_ref[1], 1.0))
        out_ref[...] = jnp.where(h >= 0, h, 0.01 * h)

    return pl.pallas_call(
        body,
        out_shape=jax.ShapeDtypeStruct((_NP, _H), jnp.float32),
    )(msg, deg)


def _h2_combine(msg2, deg, W2f, b2f, W2i, b2i):
    def body(m_ref, d_ref, wf_ref, bf_ref, wi_ref, bi_ref, out_ref):
        d0 = d_ref[0]
        d1 = d_ref[1]
        m0 = m_ref[0] / jnp.maximum(d0, 1.0)
        m1 = m_ref[1] / jnp.maximum(d1, 1.0)
        out = jnp.dot(m0, wf_ref[...], preferred_element_type=jnp.float32)
        out = out + jnp.dot(m1, wi_ref[...],
                            preferred_element_type=jnp.float32)
        out = out + jnp.minimum(d0[:, 0:1], 1.0) * bf_ref[...]
        out = out + jnp.minimum(d1[:, 0:1], 1.0) * bi_ref[...]
        out_ref[...] = out

    return pl.pallas_call(
        body,
        out_shape=jax.ShapeDtypeStruct((_NP, _C), jnp.float32),
    )(msg2, deg, W2f, b2f.reshape(1, _C), W2i, b2i.reshape(1, _C))


def kernel(x, edge_index_follows, edge_index_interacts,
           W1_f, b1_f, W1_i, b1_i, W2_f, b2_f, W2_i, b2_i):
    src_f, dst_f = edge_index_follows[0], edge_index_follows[1]
    src_i, dst_i = edge_index_interacts[0], edge_index_interacts[1]

    padz = jnp.zeros((_PADE,), jnp.int32)
    padt = jnp.full((_PADE,), _TRASH, jnp.int32)
    src1 = jnp.concatenate([src_f, padz, src_i + _N, padz]).reshape(-1, _CHUNK)
    src2 = jnp.concatenate([src_f, padz, src_i, padz]).reshape(-1, _CHUNK)
    dsts = jnp.concatenate([dst_f, padt, dst_i, padt]).reshape(-1, _CHUNK)
    zrows = jnp.zeros((_ROWS_PT, _H), jnp.float32)
    ones = jnp.ones((_CHUNK, _H), jnp.float32)

    wh = _linear1(x, W1_f, b1_f, W1_i, b1_i)
    table1 = wh.reshape(2 * _N, _H)
    msg1, deg = _AGG_L1(table1, src1, dsts, zrows, ones)
    h1p = _h1_combine(msg1, deg)
    out2 = _AGG_L2(h1p, src2, dsts, zrows, ones)
    msg2 = out2[0] if isinstance(out2, (tuple, list)) else out2
    h2p = _h2_combine(msg2, deg, W2_f, b2_f, W2_i, b2_i)
    return (h2p[:_N], h1p[:_N])


# speedup vs baseline: 28.2299x; 1.7427x over previous
import jax
import jax.numpy as jnp
from jax import lax
from jax.experimental import pallas as pl
from jax.experimental.pallas import tpu as pltpu
from jax.experimental.pallas import tpu_sc as plsc

_N = 10000
_E = 320000
_IN = 128
_H = 16
_C = 2

_CHUNK = 128
_NSUB = 16
_NCORE = 2
_CPT = 160
_EP = _CHUNK * _CPT * _NSUB
_PADE = _EP - _E
_ROWS_PT = 632
_NP = _ROWS_PT * _NSUB
_TPT = _N // _NSUB
_NBUF = 4

_SC_MESH = plsc.VectorSubcoreMesh(core_axis_name="c", subcore_axis_name="s")
_SC_PARAMS = pltpu.CompilerParams(use_tc_tiling_on_sc=False)


def _edge_loop(src_v, dst_v, tbl_s, acc, accd, ones_v, bufs, sems):
    def fire(j, buf, sem):
        pltpu.async_copy(tbl_s.at[src_v.at[j]], buf, sem)

    def wait(j, buf, sem):
        pltpu.make_async_copy(tbl_s.at[src_v.at[j]], buf, sem).wait()

    def scat(j, buf):
        pltpu.sync_copy(buf, acc.at[dst_v.at[j]], add=True)
        if accd is not None:
            pltpu.sync_copy(ones_v, accd.at[dst_v.at[j]], add=True)

    for k in range(_NBUF - 1):
        fire(k, bufs[k], sems[k])

    def step(t, carry):
        base = _NBUF * t
        for k in range(_NBUF):
            c = base + k
            kf = (k + _NBUF - 1) % _NBUF

            @pl.when(c + _NBUF - 1 < _CPT)
            def _(c=c, kf=kf):
                fire(c + _NBUF - 1, bufs[kf], sems[kf])

            wait(c, bufs[k], sems[k])
            scat(c, bufs[k])
        return carry

    lax.fori_loop(0, _CPT // _NBUF, step, 0)


def _agg_l1_body(table, src, dst, zrows, ones, msg_out, deg_out,
                 src_v, dst_v, rows0, rows1, rows2, rows3, ones_v,
                 tbl_s, acc, accd, sem0, sem1, sem2, sem3):
    cid = lax.axis_index("c")
    sid = lax.axis_index("s")
    wid = cid * _NSUB + sid
    pltpu.sync_copy(src.at[pl.ds(wid * _CPT, _CPT)], src_v)
    pltpu.sync_copy(dst.at[pl.ds(wid * _CPT, _CPT)], dst_v)
    pltpu.sync_copy(ones, ones_v)
    pltpu.sync_copy(table.at[pl.ds(cid * _N + sid * _TPT, _TPT)],
                    tbl_s.at[pl.ds(sid * _TPT, _TPT)])
    r0 = sid * _ROWS_PT
    pltpu.sync_copy(zrows, acc.at[pl.ds(r0, _ROWS_PT)])
    pltpu.sync_copy(zrows, accd.at[pl.ds(r0, _ROWS_PT)])
    plsc.subcore_barrier()

    _edge_loop(src_v, dst_v, tbl_s, acc, accd, ones_v,
               (rows0, rows1, rows2, rows3), (sem0, sem1, sem2, sem3))

    plsc.subcore_barrier()
    pltpu.sync_copy(acc.at[pl.ds(r0, _ROWS_PT)],
                    msg_out.at[cid, pl.ds(r0, _ROWS_PT)])
    pltpu.sync_copy(accd.at[pl.ds(r0, _ROWS_PT)],
                    deg_out.at[cid, pl.ds(r0, _ROWS_PT)])


_AGG_L1 = pl.kernel(
    _agg_l1_body, mesh=_SC_MESH,
    out_type=[jax.ShapeDtypeStruct((_NCORE, _NP, _H), jnp.float32)] * 2,
    scratch_types=[
        pltpu.VMEM((_CPT, _CHUNK), jnp.int32),
        pltpu.VMEM((_CPT, _CHUNK), jnp.int32),
        pltpu.VMEM((_CHUNK, _H), jnp.float32),
        pltpu.VMEM((_CHUNK, _H), jnp.float32),
        pltpu.VMEM((_CHUNK, _H), jnp.float32),
        pltpu.VMEM((_CHUNK, _H), jnp.float32),
        pltpu.VMEM((_CHUNK, _H), jnp.float32),
        pltpu.VMEM_SHARED((_N, _H), jnp.float32),
        pltpu.VMEM_SHARED((_NP, _H), jnp.float32),
        pltpu.VMEM_SHARED((_NP, _H), jnp.float32),
        pltpu.SemaphoreType.DMA,
        pltpu.SemaphoreType.DMA,
        pltpu.SemaphoreType.DMA,
        pltpu.SemaphoreType.DMA,
    ],
    compiler_params=_SC_PARAMS,
)


def _agg_l2_body(msg1, deg, src, dst, zrows, ones, msg_out, h1_out,
                 src_v, dst_v, rows0, rows1, rows2, rows3, ones_v,
                 m0_v, m1_v, d0_v, d1_v, tbl_s, acc,
                 sem0, sem1, sem2, sem3):
    cid = lax.axis_index("c")
    sid = lax.axis_index("s")
    wid = cid * _NSUB + sid
    pltpu.sync_copy(src.at[pl.ds(wid * _CPT, _CPT)], src_v)
    pltpu.sync_copy(dst.at[pl.ds(wid * _CPT, _CPT)], dst_v)
    pltpu.sync_copy(ones, ones_v)
    r0 = sid * _ROWS_PT
    pltpu.sync_copy(msg1.at[0, pl.ds(r0, _ROWS_PT)], m0_v)
    pltpu.sync_copy(msg1.at[1, pl.ds(r0, _ROWS_PT)], m1_v)
    pltpu.sync_copy(deg.at[0, pl.ds(r0, _ROWS_PT)], d0_v)
    pltpu.sync_copy(deg.at[1, pl.ds(r0, _ROWS_PT)], d1_v)

    def hbody(i, carry):
        h = (m0_v[i, :] / jnp.maximum(d0_v[i, :], 1.0)
             + m1_v[i, :] / jnp.maximum(d1_v[i, :], 1.0))
        m0_v[i, :] = jnp.where(h >= 0.0, h, h * 0.01)
        return carry

    lax.fori_loop(0, _ROWS_PT, hbody, 0)
    pltpu.sync_copy(m0_v, tbl_s.at[pl.ds(r0, _ROWS_PT)])

    @pl.when(cid == 0)
    def _():
        pltpu.sync_copy(m0_v, h1_out.at[pl.ds(r0, _ROWS_PT)])

    pltpu.sync_copy(zrows, acc.at[pl.ds(r0, _ROWS_PT)])
    plsc.subcore_barrier()

    _edge_loop(src_v, dst_v, tbl_s, acc, None, ones_v,
               (rows0, rows1, rows2, rows3), (sem0, sem1, sem2, sem3))

    plsc.subcore_barrier()
    pltpu.sync_copy(acc.at[pl.ds(r0, _ROWS_PT)],
                    msg_out.at[cid, pl.ds(r0, _ROWS_PT)])


_AGG_L2 = pl.kernel(
    _agg_l2_body, mesh=_SC_MESH,
    out_type=[jax.ShapeDtypeStruct((_NCORE, _NP, _H), jnp.float32),
              jax.ShapeDtypeStruct((_NP, _H), jnp.float32)],
    scratch_types=[
        pltpu.VMEM((_CPT, _CHUNK), jnp.int32),
        pltpu.VMEM((_CPT, _CHUNK), jnp.int32),
        pltpu.VMEM((_CHUNK, _H), jnp.float32),
        pltpu.VMEM((_CHUNK, _H), jnp.float32),
        pltpu.VMEM((_CHUNK, _H), jnp.float32),
        pltpu.VMEM((_CHUNK, _H), jnp.float32),
        pltpu.VMEM((_CHUNK, _H), jnp.float32),
        pltpu.VMEM((_ROWS_PT, _H), jnp.float32),
        pltpu.VMEM((_ROWS_PT, _H), jnp.float32),
        pltpu.VMEM((_ROWS_PT, _H), jnp.float32),
        pltpu.VMEM((_ROWS_PT, _H), jnp.float32),
        pltpu.VMEM_SHARED((_NP, _H), jnp.float32),
        pltpu.VMEM_SHARED((_NP, _H), jnp.float32),
        pltpu.SemaphoreType.DMA,
        pltpu.SemaphoreType.DMA,
        pltpu.SemaphoreType.DMA,
        pltpu.SemaphoreType.DMA,
    ],
    compiler_params=_SC_PARAMS,
)


def _linear1(x, W1f, b1f, W1i, b1i):
    blk = 1000

    def body(x_ref, wf_ref, bf_ref, wi_ref, bi_ref, out_ref):
        xb = x_ref[...]
        out_ref[0] = jnp.dot(xb, wf_ref[...],
                             preferred_element_type=jnp.float32) + bf_ref[...]
        out_ref[1] = jnp.dot(xb, wi_ref[...],
                             preferred_element_type=jnp.float32) + bi_ref[...]

    return pl.pallas_call(
        body,
        grid=(_N // blk,),
        in_specs=[
            pl.BlockSpec((blk, _IN), lambda i: (i, 0)),
            pl.BlockSpec((_IN, _H), lambda i: (0, 0)),
            pl.BlockSpec((1, _H), lambda i: (0, 0)),
            pl.BlockSpec((_IN, _H), lambda i: (0, 0)),
            pl.BlockSpec((1, _H), lambda i: (0, 0)),
        ],
        out_specs=pl.BlockSpec((2, blk, _H), lambda i: (0, i, 0)),
        out_shape=jax.ShapeDtypeStruct((2, _N, _H), jnp.float32),
    )(x, W1f, b1f.reshape(1, _H), W1i, b1i.reshape(1, _H))


def _h2_combine(msg2, deg, W2f, b2f, W2i, b2i):
    def body(m_ref, d_ref, wf_ref, bf_ref, wi_ref, bi_ref, out_ref):
        d0 = d_ref[0]
        d1 = d_ref[1]
        m0 = m_ref[0] / jnp.maximum(d0, 1.0)
        m1 = m_ref[1] / jnp.maximum(d1, 1.0)
        out = jnp.dot(m0, wf_ref[...], preferred_element_type=jnp.float32)
        out = out + jnp.dot(m1, wi_ref[...],
                            preferred_element_type=jnp.float32)
        out = out + jnp.minimum(d0[:, 0:1], 1.0) * bf_ref[...]
        out = out + jnp.minimum(d1[:, 0:1], 1.0) * bi_ref[...]
        out_ref[...] = out

    return pl.pallas_call(
        body,
        out_shape=jax.ShapeDtypeStruct((_NP, _C), jnp.float32),
    )(msg2, deg, W2f, b2f.reshape(1, _C), W2i, b2i.reshape(1, _C))


def kernel(x, edge_index_follows, edge_index_interacts,
           W1_f, b1_f, W1_i, b1_i, W2_f, b2_f, W2_i, b2_i):
    src_f, dst_f = edge_index_follows[0], edge_index_follows[1]
    src_i, dst_i = edge_index_interacts[0], edge_index_interacts[1]

    par = jnp.arange(_PADE, dtype=jnp.int32)
    padsrc = par % _N
    paddst = _N + par % (_NP - _N)
    srcs = jnp.concatenate([src_f, padsrc, src_i, padsrc]).reshape(-1, _CHUNK)
    dsts = jnp.concatenate([dst_f, paddst, dst_i, paddst]).reshape(-1, _CHUNK)
    zrows = jnp.zeros((_ROWS_PT, _H), jnp.float32)
    ones = jnp.ones((_CHUNK, _H), jnp.float32)

    wh = _linear1(x, W1_f, b1_f, W1_i, b1_i)
    table1 = wh.reshape(2 * _N, _H)
    msg1, deg = _AGG_L1(table1, srcs, dsts, zrows, ones)
    msg2, h1p = _AGG_L2(msg1, deg, srcs, dsts, zrows, ones)
    h2p = _h2_combine(msg2, deg, W2_f, b2_f, W2_i, b2_i)
    return (h2p[:_N], h1p[:_N])


# speedup vs baseline: 32.1973x; 1.1405x over previous
import jax
import jax.numpy as jnp
from jax import lax
from jax.experimental import pallas as pl
from jax.experimental.pallas import tpu as pltpu
from jax.experimental.pallas import tpu_sc as plsc

_N = 10000
_E = 320000
_IN = 128
_H = 16
_C = 2

_CHUNK = 128
_NSUB = 16
_NCORE = 2
_NCH = _E // _CHUNK
_CQ = _NCH // _NSUB
_CR = _NCH % _NSUB
_CMAX = 164
_ROWS_PT = 632
_NP = _ROWS_PT * _NSUB
_TPT = _N // _NSUB
_NBUF = 4
_NSTEP = -(-_CMAX // _NBUF)

_SC_MESH = plsc.VectorSubcoreMesh(core_axis_name="c", subcore_axis_name="s")
_SC_PARAMS = pltpu.CompilerParams(use_tc_tiling_on_sc=False)


def _chunk_range(sid):
    nct = jnp.where(sid < _CR, _CQ + 1, _CQ)
    c0 = sid * _CQ + jnp.minimum(sid, _CR)
    c0s = jnp.minimum(c0 - (c0 % 8), _NCH - _CMAX)
    return c0s, c0 - c0s, nct


def _stage_idx(ei3_f, ei3_i, cid, c0s, src_v, dst_v):
    @pl.when(cid == 0)
    def _():
        pltpu.sync_copy(ei3_f.at[0, pl.ds(c0s, _CMAX)], src_v)
        pltpu.sync_copy(ei3_f.at[1, pl.ds(c0s, _CMAX)], dst_v)

    @pl.when(cid == 1)
    def _():
        pltpu.sync_copy(ei3_i.at[0, pl.ds(c0s, _CMAX)], src_v)
        pltpu.sync_copy(ei3_i.at[1, pl.ds(c0s, _CMAX)], dst_v)


def _edge_loop(jstart, jend, src_v, dst_v, tbl_s, acc, accd, ones_v,
               bufs, sems):
    def fire(j, buf, sem):
        pltpu.async_copy(tbl_s.at[src_v.at[j]], buf, sem)

    def wait(j, buf, sem):
        pltpu.make_async_copy(tbl_s.at[src_v.at[j]], buf, sem).wait()

    def scat(j, buf):
        pltpu.sync_copy(buf, acc.at[dst_v.at[j]], add=True)
        if accd is not None:
            pltpu.sync_copy(ones_v, accd.at[dst_v.at[j]], add=True)

    for k in range(_NBUF - 1):
        fire(jstart + k, bufs[k], sems[k])

    def step(t, carry):
        base = jstart + _NBUF * t
        for k in range(_NBUF):
            c = base + k
            kf = (k + _NBUF - 1) % _NBUF

            @pl.when(c + _NBUF - 1 < jend)
            def _(c=c, kf=kf):
                fire(c + _NBUF - 1, bufs[kf], sems[kf])

            @pl.when(c < jend)
            def _(c=c, k=k):
                wait(c, bufs[k], sems[k])
                scat(c, bufs[k])
        return carry

    lax.fori_loop(0, _NSTEP, step, 0)


def _agg_l1_body(tbl_f, tbl_i, ei3_f, ei3_i, zrows, ones,
                 msg_f, msg_i, deg_f, deg_i,
                 src_v, dst_v, rows0, rows1, rows2, rows3, ones_v,
                 tbl_s, acc, accd, sem0, sem1, sem2, sem3):
    cid = lax.axis_index("c")
    sid = lax.axis_index("s")
    c0s, jstart, nct = _chunk_range(sid)
    _stage_idx(ei3_f, ei3_i, cid, c0s, src_v, dst_v)
    pltpu.sync_copy(ones, ones_v)
    ts = pl.ds(sid * _TPT, _TPT)

    @pl.when(cid == 0)
    def _():
        pltpu.sync_copy(tbl_f.at[ts], tbl_s.at[ts])

    @pl.when(cid == 1)
    def _():
        pltpu.sync_copy(tbl_i.at[ts], tbl_s.at[ts])

    r0 = sid * _ROWS_PT
    pltpu.sync_copy(zrows, acc.at[pl.ds(r0, _ROWS_PT)])
    pltpu.sync_copy(zrows, accd.at[pl.ds(r0, _ROWS_PT)])
    plsc.subcore_barrier()

    _edge_loop(jstart, jstart + nct, src_v, dst_v, tbl_s, acc, accd, ones_v,
               (rows0, rows1, rows2, rows3), (sem0, sem1, sem2, sem3))

    plsc.subcore_barrier()
    rs = pl.ds(r0, _ROWS_PT)

    @pl.when(cid == 0)
    def _():
        pltpu.sync_copy(acc.at[rs], msg_f.at[rs])
        pltpu.sync_copy(accd.at[rs], deg_f.at[rs])

    @pl.when(cid == 1)
    def _():
        pltpu.sync_copy(acc.at[rs], msg_i.at[rs])
        pltpu.sync_copy(accd.at[rs], deg_i.at[rs])


_AGG_L1 = pl.kernel(
    _agg_l1_body, mesh=_SC_MESH,
    out_type=[jax.ShapeDtypeStruct((_NP, _H), jnp.float32)] * 4,
    scratch_types=[
        pltpu.VMEM((_CMAX, _CHUNK), jnp.int32),
        pltpu.VMEM((_CMAX, _CHUNK), jnp.int32),
        pltpu.VMEM((_CHUNK, _H), jnp.float32),
        pltpu.VMEM((_CHUNK, _H), jnp.float32),
        pltpu.VMEM((_CHUNK, _H), jnp.float32),
        pltpu.VMEM((_CHUNK, _H), jnp.float32),
        pltpu.VMEM((_CHUNK, _H), jnp.float32),
        pltpu.VMEM_SHARED((_N, _H), jnp.float32),
        pltpu.VMEM_SHARED((_NP, _H), jnp.float32),
        pltpu.VMEM_SHARED((_NP, _H), jnp.float32),
        pltpu.SemaphoreType.DMA,
        pltpu.SemaphoreType.DMA,
        pltpu.SemaphoreType.DMA,
        pltpu.SemaphoreType.DMA,
    ],
    compiler_params=_SC_PARAMS,
)


def _agg_l2_body(msg_f, msg_i, deg_f, deg_i, ei3_f, ei3_i, zrows, ones,
                 msg2_f, msg2_i, h1_out,
                 src_v, dst_v, rows0, rows1, rows2, rows3, ones_v,
                 m0_v, m1_v, d0_v, d1_v, tbl_s, acc,
                 sem0, sem1, sem2, sem3):
    cid = lax.axis_index("c")
    sid = lax.axis_index("s")
    c0s, jstart, nct = _chunk_range(sid)
    _stage_idx(ei3_f, ei3_i, cid, c0s, src_v, dst_v)
    pltpu.sync_copy(ones, ones_v)
    r0 = sid * _ROWS_PT
    rs = pl.ds(r0, _ROWS_PT)
    pltpu.sync_copy(msg_f.at[rs], m0_v)
    pltpu.sync_copy(msg_i.at[rs], m1_v)
    pltpu.sync_copy(deg_f.at[rs], d0_v)
    pltpu.sync_copy(deg_i.at[rs], d1_v)

    def hbody(i, carry):
        h = (m0_v[i, :] / jnp.maximum(d0_v[i, :], 1.0)
             + m1_v[i, :] / jnp.maximum(d1_v[i, :], 1.0))
        m0_v[i, :] = jnp.where(h >= 0.0, h, h * 0.01)
        return carry

    lax.fori_loop(0, _ROWS_PT, hbody, 0)
    pltpu.sync_copy(m0_v, tbl_s.at[rs])

    @pl.when((cid == 0) & (sid < _NSUB - 1))
    def _():
        pltpu.sync_copy(m0_v, h1_out.at[rs])

    @pl.when((cid == 0) & (sid == _NSUB - 1))
    def _():
        nlast = _N - (_NSUB - 1) * _ROWS_PT
        pltpu.sync_copy(m0_v.at[pl.ds(0, nlast)],
                        h1_out.at[pl.ds((_NSUB - 1) * _ROWS_PT, nlast)])

    pltpu.sync_copy(zrows, acc.at[rs])
    plsc.subcore_barrier()

    _edge_loop(jstart, jstart + nct, src_v, dst_v, tbl_s, acc, None, ones_v,
               (rows0, rows1, rows2, rows3), (sem0, sem1, sem2, sem3))

    plsc.subcore_barrier()

    @pl.when(cid == 0)
    def _():
        pltpu.sync_copy(acc.at[rs], msg2_f.at[rs])

    @pl.when(cid == 1)
    def _():
        pltpu.sync_copy(acc.at[rs], msg2_i.at[rs])


_AGG_L2 = pl.kernel(
    _agg_l2_body, mesh=_SC_MESH,
    out_type=[jax.ShapeDtypeStruct((_NP, _H), jnp.float32),
              jax.ShapeDtypeStruct((_NP, _H), jnp.float32),
              jax.ShapeDtypeStruct((_N, _H), jnp.float32)],
    scratch_types=[
        pltpu.VMEM((_CMAX, _CHUNK), jnp.int32),
        pltpu.VMEM((_CMAX, _CHUNK), jnp.int32),
        pltpu.VMEM((_CHUNK, _H), jnp.float32),
        pltpu.VMEM((_CHUNK, _H), jnp.float32),
        pltpu.VMEM((_CHUNK, _H), jnp.float32),
        pltpu.VMEM((_CHUNK, _H), jnp.float32),
        pltpu.VMEM((_CHUNK, _H), jnp.float32),
        pltpu.VMEM((_ROWS_PT, _H), jnp.float32),
        pltpu.VMEM((_ROWS_PT, _H), jnp.float32),
        pltpu.VMEM((_ROWS_PT, _H), jnp.float32),
        pltpu.VMEM((_ROWS_PT, _H), jnp.float32),
        pltpu.VMEM_SHARED((_NP, _H), jnp.float32),
        pltpu.VMEM_SHARED((_NP, _H), jnp.float32),
        pltpu.SemaphoreType.DMA,
        pltpu.SemaphoreType.DMA,
        pltpu.SemaphoreType.DMA,
        pltpu.SemaphoreType.DMA,
    ],
    compiler_params=_SC_PARAMS,
)


def _linear1(x, W1f, b1f, W1i, b1i):
    blk = 2000

    def body(x_ref, wf_ref, bf_ref, wi_ref, bi_ref, of_ref, oi_ref):
        xb = x_ref[...]
        of_ref[...] = jnp.dot(xb, wf_ref[...],
                              preferred_element_type=jnp.float32) + bf_ref[...]
        oi_ref[...] = jnp.dot(xb, wi_ref[...],
                              preferred_element_type=jnp.float32) + bi_ref[...]

    return pl.pallas_call(
        body,
        grid=(_N // blk,),
        in_specs=[
            pl.BlockSpec((blk, _IN), lambda i: (i, 0)),
            pl.BlockSpec((_IN, _H), lambda i: (0, 0)),
            pl.BlockSpec((1, _H), lambda i: (0, 0)),
            pl.BlockSpec((_IN, _H), lambda i: (0, 0)),
            pl.BlockSpec((1, _H), lambda i: (0, 0)),
        ],
        out_specs=[pl.BlockSpec((blk, _H), lambda i: (i, 0)),
                   pl.BlockSpec((blk, _H), lambda i: (i, 0))],
        out_shape=[jax.ShapeDtypeStruct((_N, _H), jnp.float32),
                   jax.ShapeDtypeStruct((_N, _H), jnp.float32)],
    )(x, W1f, b1f.reshape(1, _H), W1i, b1i.reshape(1, _H))


def _h2_combine(msg2_f, msg2_i, deg_f, deg_i, W2f, b2f, W2i, b2i):
    blk = 2000

    def body(mf_ref, mi_ref, df_ref, di_ref, wf_ref, bf_ref, wi_ref, bi_ref,
             out_ref):
        d0 = df_ref[...]
        d1 = di_ref[...]
        m0 = mf_ref[...] / jnp.maximum(d0, 1.0)
        m1 = mi_ref[...] / jnp.maximum(d1, 1.0)
        out = jnp.dot(m0, wf_ref[...], preferred_element_type=jnp.float32)
        out = out + jnp.dot(m1, wi_ref[...],
                            preferred_element_type=jnp.float32)
        out = out + jnp.minimum(d0[:, 0:1], 1.0) * bf_ref[...]
        out = out + jnp.minimum(d1[:, 0:1], 1.0) * bi_ref[...]
        out_ref[...] = out

    return pl.pallas_call(
        body,
        grid=(_N // blk,),
        in_specs=[pl.BlockSpec((blk, _H), lambda i: (i, 0))] * 4 + [
            pl.BlockSpec((_H, _C), lambda i: (0, 0)),
            pl.BlockSpec((1, _C), lambda i: (0, 0)),
            pl.BlockSpec((_H, _C), lambda i: (0, 0)),
            pl.BlockSpec((1, _C), lambda i: (0, 0)),
        ],
        out_specs=pl.BlockSpec((blk, _C), lambda i: (i, 0)),
        out_shape=jax.ShapeDtypeStruct((_N, _C), jnp.float32),
    )(msg2_f, msg2_i, deg_f, deg_i,
      W2f, b2f.reshape(1, _C), W2i, b2i.reshape(1, _C))


def kernel(x, edge_index_follows, edge_index_interacts,
           W1_f, b1_f, W1_i, b1_i, W2_f, b2_f, W2_i, b2_i):
    ei3_f = edge_index_follows.reshape(2, _NCH, _CHUNK)
    ei3_i = edge_index_interacts.reshape(2, _NCH, _CHUNK)
    zrows = jnp.zeros((_ROWS_PT, _H), jnp.float32)
    ones = jnp.ones((_CHUNK, _H), jnp.float32)

    tbl_f, tbl_i = _linear1(x, W1_f, b1_f, W1_i, b1_i)
    msg_f, msg_i, deg_f, deg_i = _AGG_L1(tbl_f, tbl_i, ei3_f, ei3_i,
                                         zrows, ones)
    msg2_f, msg2_i, h1 = _AGG_L2(msg_f, msg_i, deg_f, deg_i,
                                 ei3_f, ei3_i, zrows, ones)
    h2 = _h2_combine(msg2_f, msg2_i, deg_f, deg_i, W2_f, b2_f, W2_i, b2_i)
    return (h2, h1)


# speedup vs baseline: 35.4251x; 1.1003x over previous
import jax
import jax.numpy as jnp
from jax import lax
from jax.experimental import pallas as pl
from jax.experimental.pallas import tpu as pltpu
from jax.experimental.pallas import tpu_sc as plsc

_N = 10000
_E = 320000
_IN = 128
_H = 16
_C = 2

_CHUNK = 128
_NSUB = 16
_NCORE = 2
_NCH = _E // _CHUNK
_CQ = _NCH // _NSUB
_CR = _NCH % _NSUB
_CMAX = 164
_ROWS_PT = 632
_NP = _ROWS_PT * _NSUB
_NBUF = 8
_NSTEP = -(-_CMAX // _NBUF)

_SC_MESH = plsc.VectorSubcoreMesh(core_axis_name="c", subcore_axis_name="s")
_SC_PARAMS = pltpu.CompilerParams(use_tc_tiling_on_sc=False)


def _chunk_range(sid):
    nct = jnp.where(sid < _CR, _CQ + 1, _CQ)
    c0 = sid * _CQ + jnp.minimum(sid, _CR)
    c0s = jnp.minimum(c0 - (c0 % 8), _NCH - _CMAX)
    return c0s, c0 - c0s, nct


def _stage_idx(ei3_f, ei3_i, cid, c0s, src_v, dst_v):
    @pl.when(cid == 0)
    def _():
        pltpu.sync_copy(ei3_f.at[0, pl.ds(c0s, _CMAX)], src_v)
        pltpu.sync_copy(ei3_f.at[1, pl.ds(c0s, _CMAX)], dst_v)

    @pl.when(cid == 1)
    def _():
        pltpu.sync_copy(ei3_i.at[0, pl.ds(c0s, _CMAX)], src_v)
        pltpu.sync_copy(ei3_i.at[1, pl.ds(c0s, _CMAX)], dst_v)


def _edge_loop(jstart, jend, src_v, dst_v, tbl, acc, accd, ones_v,
               bufs, sems):
    def fire(j, buf, sem):
        pltpu.async_copy(tbl.at[src_v.at[j]], buf, sem)

    def wait(j, buf, sem):
        pltpu.make_async_copy(tbl.at[src_v.at[j]], buf, sem).wait()

    def scat(j, buf):
        pltpu.sync_copy(buf, acc.at[dst_v.at[j]], add=True)
        if accd is not None:
            pltpu.sync_copy(ones_v, accd.at[dst_v.at[j]], add=True)

    for k in range(_NBUF - 1):
        fire(jstart + k, bufs[k], sems[k])

    def step(t, carry):
        base = jstart + _NBUF * t
        for k in range(_NBUF):
            c = base + k
            kf = (k + _NBUF - 1) % _NBUF

            @pl.when(c + _NBUF - 1 < jend)
            def _(c=c, kf=kf):
                fire(c + _NBUF - 1, bufs[kf], sems[kf])

            @pl.when(c < jend)
            def _(c=c, k=k):
                wait(c, bufs[k], sems[k])
                scat(c, bufs[k])
        return carry

    lax.fori_loop(0, _NSTEP, step, 0)


def _agg_l1_body(tbl_f, tbl_i, ei3_f, ei3_i, zrows, ones,
                 msg_f, msg_i, deg_f, deg_i,
                 src_v, dst_v, *refs):
    (b0, b1, b2, b3, b4, b5, b6, b7, ones_v, acc, accd,
     s0, s1, s2, s3, s4, s5, s6, s7) = refs
    bufs = (b0, b1, b2, b3, b4, b5, b6, b7)
    sems = (s0, s1, s2, s3, s4, s5, s6, s7)
    cid = lax.axis_index("c")
    sid = lax.axis_index("s")
    c0s, jstart, nct = _chunk_range(sid)
    _stage_idx(ei3_f, ei3_i, cid, c0s, src_v, dst_v)
    pltpu.sync_copy(ones, ones_v)
    r0 = sid * _ROWS_PT
    rs = pl.ds(r0, _ROWS_PT)
    pltpu.sync_copy(zrows, acc.at[rs])
    pltpu.sync_copy(zrows, accd.at[rs])
    plsc.subcore_barrier()

    @pl.when(cid == 0)
    def _():
        _edge_loop(jstart, jstart + nct, src_v, dst_v, tbl_f, acc, accd,
                   ones_v, bufs, sems)

    @pl.when(cid == 1)
    def _():
        _edge_loop(jstart, jstart + nct, src_v, dst_v, tbl_i, acc, accd,
                   ones_v, bufs, sems)

    plsc.subcore_barrier()

    @pl.when(cid == 0)
    def _():
        pltpu.sync_copy(acc.at[rs], msg_f.at[rs])
        pltpu.sync_copy(accd.at[rs], deg_f.at[rs])

    @pl.when(cid == 1)
    def _():
        pltpu.sync_copy(acc.at[rs], msg_i.at[rs])
        pltpu.sync_copy(accd.at[rs], deg_i.at[rs])


_AGG_L1 = pl.kernel(
    _agg_l1_body, mesh=_SC_MESH,
    out_type=[jax.ShapeDtypeStruct((_NP, _H), jnp.float32)] * 4,
    scratch_types=[
        pltpu.VMEM((_CMAX, _CHUNK), jnp.int32),
        pltpu.VMEM((_CMAX, _CHUNK), jnp.int32),
    ] + [pltpu.VMEM((_CHUNK, _H), jnp.float32)] * 9 + [
        pltpu.VMEM_SHARED((_NP, _H), jnp.float32),
        pltpu.VMEM_SHARED((_NP, _H), jnp.float32),
    ] + [pltpu.SemaphoreType.DMA] * 8,
    compiler_params=_SC_PARAMS,
)


def _agg_l2_body(msg_f, msg_i, deg_f, deg_i, ei3_f, ei3_i, zrows, ones,
                 msg2_f, msg2_i, h1_out, h1a, h1b,
                 src_v, dst_v, *refs):
    (b0, b1, b2, b3, b4, b5, b6, b7, ones_v,
     m0_v, m1_v, d0_v, d1_v, acc,
     s0, s1, s2, s3, s4, s5, s6, s7) = refs
    bufs = (b0, b1, b2, b3, b4, b5, b6, b7)
    sems = (s0, s1, s2, s3, s4, s5, s6, s7)
    cid = lax.axis_index("c")
    sid = lax.axis_index("s")
    c0s, jstart, nct = _chunk_range(sid)
    _stage_idx(ei3_f, ei3_i, cid, c0s, src_v, dst_v)
    pltpu.sync_copy(ones, ones_v)
    r0 = sid * _ROWS_PT
    rs = pl.ds(r0, _ROWS_PT)
    pltpu.sync_copy(msg_f.at[rs], m0_v)
    pltpu.sync_copy(msg_i.at[rs], m1_v)
    pltpu.sync_copy(deg_f.at[rs], d0_v)
    pltpu.sync_copy(deg_i.at[rs], d1_v)

    def hbody(i, carry):
        h = (m0_v[i, :] / jnp.maximum(d0_v[i, :], 1.0)
             + m1_v[i, :] / jnp.maximum(d1_v[i, :], 1.0))
        m0_v[i, :] = jnp.where(h >= 0.0, h, h * 0.01)
        return carry

    lax.fori_loop(0, _ROWS_PT, hbody, 0)

    @pl.when(cid == 0)
    def _():
        pltpu.sync_copy(m0_v, h1a.at[rs])

    @pl.when(cid == 1)
    def _():
        pltpu.sync_copy(m0_v, h1b.at[rs])

    @pl.when((cid == 0) & (sid < _NSUB - 1))
    def _():
        pltpu.sync_copy(m0_v, h1_out.at[rs])

    @pl.when((cid == 0) & (sid == _NSUB - 1))
    def _():
        nlast = _N - (_NSUB - 1) * _ROWS_PT
        pltpu.sync_copy(m0_v.at[pl.ds(0, nlast)],
                        h1_out.at[pl.ds((_NSUB - 1) * _ROWS_PT, nlast)])

    pltpu.sync_copy(zrows, acc.at[rs])
    plsc.subcore_barrier()

    @pl.when(cid == 0)
    def _():
        _edge_loop(jstart, jstart + nct, src_v, dst_v, h1a, acc, None,
                   ones_v, bufs, sems)

    @pl.when(cid == 1)
    def _():
        _edge_loop(jstart, jstart + nct, src_v, dst_v, h1b, acc, None,
                   ones_v, bufs, sems)

    plsc.subcore_barrier()

    @pl.when(cid == 0)
    def _():
        pltpu.sync_copy(acc.at[rs], msg2_f.at[rs])

    @pl.when(cid == 1)
    def _():
        pltpu.sync_copy(acc.at[rs], msg2_i.at[rs])


_AGG_L2 = pl.kernel(
    _agg_l2_body, mesh=_SC_MESH,
    out_type=[jax.ShapeDtypeStruct((_NP, _H), jnp.float32),
              jax.ShapeDtypeStruct((_NP, _H), jnp.float32),
              jax.ShapeDtypeStruct((_N, _H), jnp.float32),
              jax.ShapeDtypeStruct((_NP, _H), jnp.float32),
              jax.ShapeDtypeStruct((_NP, _H), jnp.float32)],
    scratch_types=[
        pltpu.VMEM((_CMAX, _CHUNK), jnp.int32),
        pltpu.VMEM((_CMAX, _CHUNK), jnp.int32),
    ] + [pltpu.VMEM((_CHUNK, _H), jnp.float32)] * 9 + [
        pltpu.VMEM((_ROWS_PT, _H), jnp.float32),
        pltpu.VMEM((_ROWS_PT, _H), jnp.float32),
        pltpu.VMEM((_ROWS_PT, _H), jnp.float32),
        pltpu.VMEM((_ROWS_PT, _H), jnp.float32),
        pltpu.VMEM_SHARED((_NP, _H), jnp.float32),
    ] + [pltpu.SemaphoreType.DMA] * 8,
    compiler_params=_SC_PARAMS,
)


def _linear1(x, W1f, b1f, W1i, b1i):
    blk = 2000

    def body(x_ref, wf_ref, bf_ref, wi_ref, bi_ref, of_ref, oi_ref):
        xb = x_ref[...]
        of_ref[...] = jnp.dot(xb, wf_ref[...],
                              preferred_element_type=jnp.float32) + bf_ref[...]
        oi_ref[...] = jnp.dot(xb, wi_ref[...],
                              preferred_element_type=jnp.float32) + bi_ref[...]

    return pl.pallas_call(
        body,
        grid=(_N // blk,),
        in_specs=[
            pl.BlockSpec((blk, _IN), lambda i: (i, 0)),
            pl.BlockSpec((_IN, _H), lambda i: (0, 0)),
            pl.BlockSpec((1, _H), lambda i: (0, 0)),
            pl.BlockSpec((_IN, _H), lambda i: (0, 0)),
            pl.BlockSpec((1, _H), lambda i: (0, 0)),
        ],
        out_specs=[pl.BlockSpec((blk, _H), lambda i: (i, 0)),
                   pl.BlockSpec((blk, _H), lambda i: (i, 0))],
        out_shape=[jax.ShapeDtypeStruct((_N, _H), jnp.float32),
                   jax.ShapeDtypeStruct((_N, _H), jnp.float32)],
    )(x, W1f, b1f.reshape(1, _H), W1i, b1i.reshape(1, _H))


def _h2_combine(msg2_f, msg2_i, deg_f, deg_i, W2f, b2f, W2i, b2i):
    blk = 2000

    def body(mf_ref, mi_ref, df_ref, di_ref, wf_ref, bf_ref, wi_ref, bi_ref,
             out_ref):
        d0 = df_ref[...]
        d1 = di_ref[...]
        m0 = mf_ref[...] / jnp.maximum(d0, 1.0)
        m1 = mi_ref[...] / jnp.maximum(d1, 1.0)
        out = jnp.dot(m0, wf_ref[...], preferred_element_type=jnp.float32)
        out = out + jnp.dot(m1, wi_ref[...],
                            preferred_element_type=jnp.float32)
        out = out + jnp.minimum(d0[:, 0:1], 1.0) * bf_ref[...]
        out = out + jnp.minimum(d1[:, 0:1], 1.0) * bi_ref[...]
        out_ref[...] = out

    return pl.pallas_call(
        body,
        grid=(_N // blk,),
        in_specs=[pl.BlockSpec((blk, _H), lambda i: (i, 0))] * 4 + [
            pl.BlockSpec((_H, _C), lambda i: (0, 0)),
            pl.BlockSpec((1, _C), lambda i: (0, 0)),
            pl.BlockSpec((_H, _C), lambda i: (0, 0)),
            pl.BlockSpec((1, _C), lambda i: (0, 0)),
        ],
        out_specs=pl.BlockSpec((blk, _C), lambda i: (i, 0)),
        out_shape=jax.ShapeDtypeStruct((_N, _C), jnp.float32),
    )(msg2_f, msg2_i, deg_f, deg_i,
      W2f, b2f.reshape(1, _C), W2i, b2i.reshape(1, _C))


def kernel(x, edge_index_follows, edge_index_interacts,
           W1_f, b1_f, W1_i, b1_i, W2_f, b2_f, W2_i, b2_i):
    ei3_f = edge_index_follows.reshape(2, _NCH, _CHUNK)
    ei3_i = edge_index_interacts.reshape(2, _NCH, _CHUNK)
    zrows = jnp.zeros((_ROWS_PT, _H), jnp.float32)
    ones = jnp.ones((_CHUNK, _H), jnp.float32)

    tbl_f, tbl_i = _linear1(x, W1_f, b1_f, W1_i, b1_i)
    msg_f, msg_i, deg_f, deg_i = _AGG_L1(tbl_f, tbl_i, ei3_f, ei3_i,
                                         zrows, ones)
    msg2_f, msg2_i, h1, _, _ = _AGG_L2(msg_f, msg_i, deg_f, deg_i,
                                       ei3_f, ei3_i, zrows, ones)
    h2 = _h2_combine(msg2_f, msg2_i, deg_f, deg_i, W2_f, b2_f, W2_i, b2_i)
    return (h2, h1)


# speedup vs baseline: 38.6561x; 1.0912x over previous
import jax
import jax.numpy as jnp
from jax import lax
from jax.experimental import pallas as pl
from jax.experimental.pallas import tpu as pltpu
from jax.experimental.pallas import tpu_sc as plsc

_N = 10000
_E = 320000
_IN = 128
_H = 16
_C = 2

_CHUNK = 128
_NSUB = 16
_NCORE = 2
_NCH = _E // _CHUNK
_CQ = _NCH // _NSUB
_CR = _NCH % _NSUB
_CMAX = 164
_ROWS_PT = 632
_NP = _ROWS_PT * _NSUB
_NBUF = 8
_NSTEP = -(-_CMAX // _NBUF)

_SC_MESH = plsc.VectorSubcoreMesh(core_axis_name="c", subcore_axis_name="s")
_SC_PARAMS = pltpu.CompilerParams(use_tc_tiling_on_sc=False)


def _chunk_range(sid):
    nct = jnp.where(sid < _CR, _CQ + 1, _CQ)
    c0 = sid * _CQ + jnp.minimum(sid, _CR)
    c0s = jnp.minimum(c0 - (c0 % 8), _NCH - _CMAX)
    return c0s, c0 - c0s, nct


def _edge_loop(jstart, jend, src_v, dst_v, tbl, acc, accd, ones_v,
               bufs, sems, ssems):
    def fire(j, buf, sem):
        pltpu.async_copy(tbl.at[src_v.at[j]], buf, sem)

    def wait(j, buf, sem):
        pltpu.make_async_copy(tbl.at[src_v.at[j]], buf, sem).wait()

    def wait_scat(j, buf, ssem):
        pltpu.make_async_copy(buf, acc.at[dst_v.at[j]], ssem).wait()

    def scat(j, buf, ssem):
        pltpu.async_copy(buf, acc.at[dst_v.at[j]], ssem, add=True)
        if accd is not None:
            pltpu.sync_copy(ones_v, accd.at[dst_v.at[j]], add=True)

    for k in range(_NBUF - 1):
        fire(jstart + k, bufs[k], sems[k])

    def step(t, carry):
        base = jstart + _NBUF * t
        for k in range(_NBUF):
            c = base + k
            kf = (k + _NBUF - 1) % _NBUF

            @pl.when(c + _NBUF - 1 < jend)
            def _(c=c, k=k, kf=kf):
                @pl.when(c - 1 >= jstart)
                def _():
                    wait_scat(c - 1, bufs[kf], ssems[kf])

                fire(c + _NBUF - 1, bufs[kf], sems[kf])

            @pl.when(c < jend)
            def _(c=c, k=k):
                wait(c, bufs[k], sems[k])
                scat(c, bufs[k], ssems[k])
        return carry

    lax.fori_loop(0, _NSTEP, step, 0)
    for k in range(_NBUF):
        wait_scat(jend - 1, bufs[k], ssems[k])


def _agg_l1_body(tbl_f, tbl_i, ei3_f, ei3_i, zrows, ones,
                 msg_f, msg_i, deg_f, deg_i,
                 src_v, dst_v, *refs):
    (b0, b1, b2, b3, b4, b5, b6, b7, ones_v, acc, accd,
     s0, s1, s2, s3, s4, s5, s6, s7,
     t0, t1, t2, t3, t4, t5, t6, t7) = refs
    bufs = (b0, b1, b2, b3, b4, b5, b6, b7)
    sems = (s0, s1, s2, s3, s4, s5, s6, s7)
    ssems = (t0, t1, t2, t3, t4, t5, t6, t7)
    cid = lax.axis_index("c")
    sid = lax.axis_index("s")
    c0s, jstart, nct = _chunk_range(sid)
    r0 = sid * _ROWS_PT
    rs = pl.ds(r0, _ROWS_PT)
    @pl.when(cid == 0)
    def _():
        pltpu.async_copy(ei3_f.at[0, pl.ds(c0s, _CMAX)], src_v, t0)
        pltpu.async_copy(ei3_f.at[1, pl.ds(c0s, _CMAX)], dst_v, t1)

    @pl.when(cid == 1)
    def _():
        pltpu.async_copy(ei3_i.at[0, pl.ds(c0s, _CMAX)], src_v, t0)
        pltpu.async_copy(ei3_i.at[1, pl.ds(c0s, _CMAX)], dst_v, t1)

    d3 = pltpu.async_copy(ones, ones_v, t2)
    d4 = pltpu.async_copy(zrows, acc.at[rs], t3)
    d5 = pltpu.async_copy(zrows, accd.at[rs], t4)
    pltpu.make_async_copy(ei3_f.at[0, pl.ds(c0s, _CMAX)], src_v, t0).wait()
    pltpu.make_async_copy(ei3_f.at[1, pl.ds(c0s, _CMAX)], dst_v, t1).wait()
    d3.wait()
    d4.wait()
    d5.wait()
    plsc.subcore_barrier()

    @pl.when(cid == 0)
    def _():
        _edge_loop(jstart, jstart + nct, src_v, dst_v, tbl_f, acc, accd,
                   ones_v, bufs, sems, ssems)

    @pl.when(cid == 1)
    def _():
        _edge_loop(jstart, jstart + nct, src_v, dst_v, tbl_i, acc, accd,
                   ones_v, bufs, sems, ssems)

    plsc.subcore_barrier()

    @pl.when(cid == 0)
    def _():
        pltpu.sync_copy(acc.at[rs], msg_f.at[rs])
        pltpu.sync_copy(accd.at[rs], deg_f.at[rs])

    @pl.when(cid == 1)
    def _():
        pltpu.sync_copy(acc.at[rs], msg_i.at[rs])
        pltpu.sync_copy(accd.at[rs], deg_i.at[rs])


_AGG_L1 = pl.kernel(
    _agg_l1_body, mesh=_SC_MESH,
    out_type=[jax.ShapeDtypeStruct((_NP, _H), jnp.float32)] * 4,
    scratch_types=[
        pltpu.VMEM((_CMAX, _CHUNK), jnp.int32),
        pltpu.VMEM((_CMAX, _CHUNK), jnp.int32),
    ] + [pltpu.VMEM((_CHUNK, _H), jnp.float32)] * 9 + [
        pltpu.VMEM_SHARED((_NP, _H), jnp.float32),
        pltpu.VMEM_SHARED((_NP, _H), jnp.float32),
    ] + [pltpu.SemaphoreType.DMA] * 16,
    compiler_params=_SC_PARAMS,
)


def _agg_l2_body(msg_f, msg_i, deg_f, deg_i, ei3_f, ei3_i, zrows, ones,
                 msg2_f, msg2_i, h1_out, h1a, h1b,
                 src_v, dst_v, *refs):
    (b0, b1, b2, b3, b4, b5, b6, b7, ones_v,
     m0_v, m1_v, d0_v, d1_v, acc,
     s0, s1, s2, s3, s4, s5, s6, s7,
     t0, t1, t2, t3, t4, t5, t6, t7) = refs
    bufs = (b0, b1, b2, b3, b4, b5, b6, b7)
    sems = (s0, s1, s2, s3, s4, s5, s6, s7)
    ssems = (t0, t1, t2, t3, t4, t5, t6, t7)
    cid = lax.axis_index("c")
    sid = lax.axis_index("s")
    c0s, jstart, nct = _chunk_range(sid)
    r0 = sid * _ROWS_PT
    rs = pl.ds(r0, _ROWS_PT)

    @pl.when(cid == 0)
    def _():
        pltpu.async_copy(ei3_f.at[0, pl.ds(c0s, _CMAX)], src_v, t0)
        pltpu.async_copy(ei3_f.at[1, pl.ds(c0s, _CMAX)], dst_v, t1)

    @pl.when(cid == 1)
    def _():
        pltpu.async_copy(ei3_i.at[0, pl.ds(c0s, _CMAX)], src_v, t0)
        pltpu.async_copy(ei3_i.at[1, pl.ds(c0s, _CMAX)], dst_v, t1)

    d2 = pltpu.async_copy(ones, ones_v, t2)
    d3 = pltpu.async_copy(msg_f.at[rs], m0_v, t3)
    d4 = pltpu.async_copy(msg_i.at[rs], m1_v, t4)
    d5 = pltpu.async_copy(deg_f.at[rs], d0_v, t5)
    d6 = pltpu.async_copy(deg_i.at[rs], d1_v, t6)
    d7 = pltpu.async_copy(zrows, acc.at[rs], t7)
    pltpu.make_async_copy(ei3_f.at[0, pl.ds(c0s, _CMAX)], src_v, t0).wait()
    pltpu.make_async_copy(ei3_f.at[1, pl.ds(c0s, _CMAX)], dst_v, t1).wait()
    d2.wait()
    d3.wait()
    d4.wait()
    d5.wait()
    d6.wait()
    d7.wait()

    def hbody(i, carry):
        h = (m0_v[i, :] / jnp.maximum(d0_v[i, :], 1.0)
             + m1_v[i, :] / jnp.maximum(d1_v[i, :], 1.0))
        m0_v[i, :] = jnp.where(h >= 0.0, h, h * 0.01)
        return carry

    lax.fori_loop(0, _ROWS_PT, hbody, 0)

    @pl.when(cid == 0)
    def _():
        pltpu.sync_copy(m0_v, h1a.at[rs])

    @pl.when(cid == 1)
    def _():
        pltpu.sync_copy(m0_v, h1b.at[rs])

    @pl.when((cid == 0) & (sid < _NSUB - 1))
    def _():
        pltpu.sync_copy(m0_v, h1_out.at[rs])

    @pl.when((cid == 0) & (sid == _NSUB - 1))
    def _():
        nlast = _N - (_NSUB - 1) * _ROWS_PT
        pltpu.sync_copy(m0_v.at[pl.ds(0, nlast)],
                        h1_out.at[pl.ds((_NSUB - 1) * _ROWS_PT, nlast)])

    plsc.subcore_barrier()

    @pl.when(cid == 0)
    def _():
        _edge_loop(jstart, jstart + nct, src_v, dst_v, h1a, acc, None,
                   ones_v, bufs, sems, ssems)

    @pl.when(cid == 1)
    def _():
        _edge_loop(jstart, jstart + nct, src_v, dst_v, h1b, acc, None,
                   ones_v, bufs, sems, ssems)

    plsc.subcore_barrier()

    @pl.when(cid == 0)
    def _():
        pltpu.sync_copy(acc.at[rs], msg2_f.at[rs])

    @pl.when(cid == 1)
    def _():
        pltpu.sync_copy(acc.at[rs], msg2_i.at[rs])


_AGG_L2 = pl.kernel(
    _agg_l2_body, mesh=_SC_MESH,
    out_type=[jax.ShapeDtypeStruct((_NP, _H), jnp.float32),
              jax.ShapeDtypeStruct((_NP, _H), jnp.float32),
              jax.ShapeDtypeStruct((_N, _H), jnp.float32),
              jax.ShapeDtypeStruct((_NP, _H), jnp.float32),
              jax.ShapeDtypeStruct((_NP, _H), jnp.float32)],
    scratch_types=[
        pltpu.VMEM((_CMAX, _CHUNK), jnp.int32),
        pltpu.VMEM((_CMAX, _CHUNK), jnp.int32),
    ] + [pltpu.VMEM((_CHUNK, _H), jnp.float32)] * 9 + [
        pltpu.VMEM((_ROWS_PT, _H), jnp.float32),
        pltpu.VMEM((_ROWS_PT, _H), jnp.float32),
        pltpu.VMEM((_ROWS_PT, _H), jnp.float32),
        pltpu.VMEM((_ROWS_PT, _H), jnp.float32),
        pltpu.VMEM_SHARED((_NP, _H), jnp.float32),
    ] + [pltpu.SemaphoreType.DMA] * 16,
    compiler_params=_SC_PARAMS,
)


def _linear1(x, W1f, b1f, W1i, b1i):
    blk = 2000

    def body(x_ref, wf_ref, bf_ref, wi_ref, bi_ref, of_ref, oi_ref):
        xb = x_ref[...]
        of_ref[...] = jnp.dot(xb, wf_ref[...],
                              preferred_element_type=jnp.float32) + bf_ref[...]
        oi_ref[...] = jnp.dot(xb, wi_ref[...],
                              preferred_element_type=jnp.float32) + bi_ref[...]

    return pl.pallas_call(
        body,
        grid=(_N // blk,),
        in_specs=[
            pl.BlockSpec((blk, _IN), lambda i: (i, 0)),
            pl.BlockSpec((_IN, _H), lambda i: (0, 0)),
            pl.BlockSpec((1, _H), lambda i: (0, 0)),
            pl.BlockSpec((_IN, _H), lambda i: (0, 0)),
            pl.BlockSpec((1, _H), lambda i: (0, 0)),
        ],
        out_specs=[pl.BlockSpec((blk, _H), lambda i: (i, 0)),
                   pl.BlockSpec((blk, _H), lambda i: (i, 0))],
        out_shape=[jax.ShapeDtypeStruct((_N, _H), jnp.float32),
                   jax.ShapeDtypeStruct((_N, _H), jnp.float32)],
    )(x, W1f, b1f.reshape(1, _H), W1i, b1i.reshape(1, _H))


def _h2_combine(msg2_f, msg2_i, deg_f, deg_i, W2f, b2f, W2i, b2i):
    blk = 2000

    def body(mf_ref, mi_ref, df_ref, di_ref, wf_ref, bf_ref, wi_ref, bi_ref,
             out_ref):
        d0 = df_ref[...]
        d1 = di_ref[...]
        m0 = mf_ref[...] / jnp.maximum(d0, 1.0)
        m1 = mi_ref[...] / jnp.maximum(d1, 1.0)
        out = jnp.dot(m0, wf_ref[...], preferred_element_type=jnp.float32)
        out = out + jnp.dot(m1, wi_ref[...],
                            preferred_element_type=jnp.float32)
        out = out + jnp.minimum(d0[:, 0:1], 1.0) * bf_ref[...]
        out = out + jnp.minimum(d1[:, 0:1], 1.0) * bi_ref[...]
        out_ref[...] = out

    return pl.pallas_call(
        body,
        grid=(_N // blk,),
        in_specs=[pl.BlockSpec((blk, _H), lambda i: (i, 0))] * 4 + [
            pl.BlockSpec((_H, _C), lambda i: (0, 0)),
            pl.BlockSpec((1, _C), lambda i: (0, 0)),
            pl.BlockSpec((_H, _C), lambda i: (0, 0)),
            pl.BlockSpec((1, _C), lambda i: (0, 0)),
        ],
        out_specs=pl.BlockSpec((blk, _C), lambda i: (i, 0)),
        out_shape=jax.ShapeDtypeStruct((_N, _C), jnp.float32),
    )(msg2_f, msg2_i, deg_f, deg_i,
      W2f, b2f.reshape(1, _C), W2i, b2i.reshape(1, _C))


def kernel(x, edge_index_follows, edge_index_interacts,
           W1_f, b1_f, W1_i, b1_i, W2_f, b2_f, W2_i, b2_i):
    ei3_f = edge_index_follows.reshape(2, _NCH, _CHUNK)
    ei3_i = edge_index_interacts.reshape(2, _NCH, _CHUNK)
    zrows = jnp.zeros((_ROWS_PT, _H), jnp.float32)
    ones = jnp.ones((_CHUNK, _H), jnp.float32)

    tbl_f, tbl_i = _linear1(x, W1_f, b1_f, W1_i, b1_i)
    msg_f, msg_i, deg_f, deg_i = _AGG_L1(tbl_f, tbl_i, ei3_f, ei3_i,
                                         zrows, ones)
    msg2_f, msg2_i, h1, _, _ = _AGG_L2(msg_f, msg_i, deg_f, deg_i,
                                       ei3_f, ei3_i, zrows, ones)
    h2 = _h2_combine(msg2_f, msg2_i, deg_f, deg_i, W2_f, b2_f, W2_i, b2_i)
    return (h2, h1)
